# baseline (speedup 1.0000x reference)
import functools

import jax
import jax.numpy as jnp
from jax import lax
from jax.experimental import pallas as pl
from jax.experimental.pallas import tpu as pltpu
from jax.experimental.pallas import tpu_sc as plsc

D = 128
NC = 2
NS = 16
NW = NC * NS
CHUNK = 512
LANES = 16


def _sc_body(pe_t_h, pe_h_h, pe_w_h, t_h, h_h, w_h, out_h,
             pt_v, ph_v, pw_v, ti_v, hi_v, wi_v, ob_v):
    wid = lax.axis_index("s") * NC + lax.axis_index("c")
    n = t_h.shape[0]
    per_w = n // NW
    base = wid * per_w

    pltpu.sync_copy(pe_t_h, pt_v)
    pltpu.sync_copy(pe_h_h, ph_v)
    pltpu.sync_copy(pe_w_h, pw_v)

    lane_iota = lax.iota(jnp.int32, LANES)

    def chunk_body(i, carry):
        off = base + i * CHUNK
        pltpu.sync_copy(t_h.at[pl.ds(off, CHUNK)], ti_v)
        pltpu.sync_copy(h_h.at[pl.ds(off, CHUNK)], hi_v)
        pltpu.sync_copy(w_h.at[pl.ds(off, CHUNK)], wi_v)

        @plsc.parallel_loop(0, CHUNK // LANES, 1)
        def group_body(g):
            tids = ti_v[pl.ds(g * LANES, LANES)]
            hids = hi_v[pl.ds(g * LANES, LANES)]
            wids = wi_v[pl.ds(g * LANES, LANES)]
            rows = lane_iota + g * LANES

            @plsc.parallel_loop(0, D, 1, unroll=8)
            def col_body(c):
                colv = jnp.full((LANES,), 0, jnp.int32) + c
                v = (plsc.load_gather(pt_v, [tids, colv])
                     + plsc.load_gather(ph_v, [hids, colv])
                     + plsc.load_gather(pw_v, [wids, colv]))
                plsc.store_scatter(ob_v, [rows, colv], v)
        pltpu.sync_copy(ob_v, out_h.at[pl.ds(off, CHUNK)])
        return carry

    lax.fori_loop(0, per_w // CHUNK, chunk_body, 0)


def kernel(pe_t, pe_h, pe_w, t, h, w):
    b, l = t.shape
    n = b * l
    tf = t.reshape(n)
    hf = h.reshape(n)
    wf = w.reshape(n)
    mesh = plsc.VectorSubcoreMesh(core_axis_name="c", subcore_axis_name="s")
    run = pl.kernel(
        _sc_body,
        mesh=mesh,
        compiler_params=pltpu.CompilerParams(needs_layout_passes=False),
        out_type=jax.ShapeDtypeStruct((n, D), jnp.float32),
        scratch_types=[
            pltpu.VMEM((pe_t.shape[0], D), jnp.float32),
            pltpu.VMEM((pe_h.shape[0], D), jnp.float32),
            pltpu.VMEM((pe_w.shape[0], D), jnp.float32),
            pltpu.VMEM((CHUNK,), jnp.int32),
            pltpu.VMEM((CHUNK,), jnp.int32),
            pltpu.VMEM((CHUNK,), jnp.int32),
            pltpu.VMEM((CHUNK, D), jnp.float32),
        ],
    )
    out = run(pe_t, pe_h, pe_w, tf, hf, wf)
    return out.reshape(b, l, D)


# speedup vs baseline: 6.3441x; 6.3441x over previous
import functools

import jax
import jax.numpy as jnp
from jax import lax
from jax.experimental import pallas as pl
from jax.experimental.pallas import tpu as pltpu
from jax.experimental.pallas import tpu_sc as plsc

D = 128
NC = 2
NS = 16
NW = NC * NS
CHUNK = 512
LANES = 16


def _sc_body(pe_t_h, pe_h_h, pe_w_h, t_h, h_h, w_h, out_h,
             pt_v, ph_v, pw_v, ti_v, hi_v, wi_v, ob_v):
    wid = lax.axis_index("s") * NC + lax.axis_index("c")
    n = t_h.shape[0]
    per_w = n // NW
    base = wid * per_w

    pltpu.sync_copy(pe_t_h, pt_v)
    pltpu.sync_copy(pe_h_h, ph_v)
    pltpu.sync_copy(pe_w_h, pw_v)

    lane_iota = lax.iota(jnp.int32, LANES)

    def chunk_body(i, carry):
        off = base + i * CHUNK
        pltpu.sync_copy(t_h.at[pl.ds(off, CHUNK)], ti_v)
        pltpu.sync_copy(h_h.at[pl.ds(off, CHUNK)], hi_v)
        pltpu.sync_copy(w_h.at[pl.ds(off, CHUNK)], wi_v)

        @plsc.parallel_loop(0, CHUNK // LANES, 1)
        def group_body(g):
            tids = ti_v[pl.ds(g * LANES, LANES)]
            hids = hi_v[pl.ds(g * LANES, LANES)]
            wids = wi_v[pl.ds(g * LANES, LANES)]
            for j in range(LANES):
                tr = tids[j]
                hr = hids[j]
                wr = wids[j]
                r = g * LANES + j
                for c in range(D // LANES):
                    v = (pt_v[tr, pl.ds(c * LANES, LANES)]
                         + ph_v[hr, pl.ds(c * LANES, LANES)]
                         + pw_v[wr, pl.ds(c * LANES, LANES)])
                    ob_v[r, pl.ds(c * LANES, LANES)] = v
        pltpu.sync_copy(ob_v, out_h.at[pl.ds(off, CHUNK)])
        return carry

    lax.fori_loop(0, per_w // CHUNK, chunk_body, 0)


def kernel(pe_t, pe_h, pe_w, t, h, w):
    b, l = t.shape
    n = b * l
    tf = t.reshape(n)
    hf = h.reshape(n)
    wf = w.reshape(n)
    mesh = plsc.VectorSubcoreMesh(core_axis_name="c", subcore_axis_name="s")
    run = pl.kernel(
        _sc_body,
        mesh=mesh,
        compiler_params=pltpu.CompilerParams(needs_layout_passes=False),
        out_type=jax.ShapeDtypeStruct((n, D), jnp.float32),
        scratch_types=[
            pltpu.VMEM((pe_t.shape[0], D), jnp.float32),
            pltpu.VMEM((pe_h.shape[0], D), jnp.float32),
            pltpu.VMEM((pe_w.shape[0], D), jnp.float32),
            pltpu.VMEM((CHUNK,), jnp.int32),
            pltpu.VMEM((CHUNK,), jnp.int32),
            pltpu.VMEM((CHUNK,), jnp.int32),
            pltpu.VMEM((CHUNK, D), jnp.float32),
        ],
    )
    out = run(pe_t, pe_h, pe_w, tf, hf, wf)
    return out.reshape(b, l, D)


# speedup vs baseline: 6.5163x; 1.0271x over previous
import functools

import jax
import jax.numpy as jnp
from jax import lax
from jax.experimental import pallas as pl
from jax.experimental.pallas import tpu as pltpu
from jax.experimental.pallas import tpu_sc as plsc

D = 128
NC = 2
NS = 16
NW = NC * NS
CHUNK = 256
NBUF = 2
LANES = 16


def _sc_body(pe_t_h, pe_h_h, pe_w_h, t_h, h_h, w_h, out_h,
             pt_v, ph_v, pw_v, ti_v, hi_v, wi_v, ob_v,
             sit, sih, siw, so):
    wid = lax.axis_index("s") * NC + lax.axis_index("c")
    n = t_h.shape[0]
    per_w = n // NW
    base = wid * per_w
    nch = per_w // CHUNK

    pltpu.sync_copy(pe_t_h, pt_v)
    pltpu.sync_copy(pe_h_h, ph_v)
    pltpu.sync_copy(pe_w_h, pw_v)

    def idx_copies(k, slot):
        off = base + k * CHUNK
        return (
            pltpu.make_async_copy(t_h.at[pl.ds(off, CHUNK)],
                                  ti_v.at[pl.ds(slot * CHUNK, CHUNK)],
                                  sit.at[slot]),
            pltpu.make_async_copy(h_h.at[pl.ds(off, CHUNK)],
                                  hi_v.at[pl.ds(slot * CHUNK, CHUNK)],
                                  sih.at[slot]),
            pltpu.make_async_copy(w_h.at[pl.ds(off, CHUNK)],
                                  wi_v.at[pl.ds(slot * CHUNK, CHUNK)],
                                  siw.at[slot]),
        )

    def out_copy(k, slot):
        off = base + k * CHUNK
        return pltpu.make_async_copy(
            ob_v.at[pl.ds(slot * CHUNK, CHUNK)],
            out_h.at[pl.ds(off, CHUNK)],
            so.at[slot])

    for k0 in range(NBUF):
        for cp in idx_copies(k0, k0):
            cp.start()

    def chunk_body(k, carry):
        slot = k % NBUF
        for cp in idx_copies(k, slot):
            cp.wait()

        @pl.when(k >= NBUF)
        def _():
            out_copy(k - NBUF, slot).wait()

        rbase = slot * CHUNK
        ibase = slot * CHUNK

        @plsc.parallel_loop(0, CHUNK // LANES, 1)
        def group_body(g):
            tids = ti_v[pl.ds(ibase + g * LANES, LANES)]
            hids = hi_v[pl.ds(ibase + g * LANES, LANES)]
            wids = wi_v[pl.ds(ibase + g * LANES, LANES)]
            for j in range(LANES):
                tr = tids[j]
                hr = hids[j]
                wr = wids[j]
                r = rbase + g * LANES + j
                for c in range(D // LANES):
                    v = (pt_v[tr, pl.ds(c * LANES, LANES)]
                         + ph_v[hr, pl.ds(c * LANES, LANES)]
                         + pw_v[wr, pl.ds(c * LANES, LANES)])
                    ob_v[r, pl.ds(c * LANES, LANES)] = v

        out_copy(k, slot).start()

        @pl.when(k + NBUF < nch)
        def _():
            for cp in idx_copies(k + NBUF, slot):
                cp.start()

        return carry

    lax.fori_loop(0, nch, chunk_body, 0)

    for k0 in range(nch - NBUF, nch):
        out_copy(k0, k0 % NBUF).wait()


def kernel(pe_t, pe_h, pe_w, t, h, w):
    b, l = t.shape
    n = b * l
    tf = t.reshape(n)
    hf = h.reshape(n)
    wf = w.reshape(n)
    mesh = plsc.VectorSubcoreMesh(core_axis_name="c", subcore_axis_name="s")
    run = pl.kernel(
        _sc_body,
        mesh=mesh,
        compiler_params=pltpu.CompilerParams(needs_layout_passes=False),
        out_type=jax.ShapeDtypeStruct((n, D), jnp.float32),
        scratch_types=[
            pltpu.VMEM((pe_t.shape[0], D), jnp.float32),
            pltpu.VMEM((pe_h.shape[0], D), jnp.float32),
            pltpu.VMEM((pe_w.shape[0], D), jnp.float32),
            pltpu.VMEM((NBUF * CHUNK,), jnp.int32),
            pltpu.VMEM((NBUF * CHUNK,), jnp.int32),
            pltpu.VMEM((NBUF * CHUNK,), jnp.int32),
            pltpu.VMEM((NBUF * CHUNK, D), jnp.float32),
            pltpu.SemaphoreType.DMA((NBUF,)),
            pltpu.SemaphoreType.DMA((NBUF,)),
            pltpu.SemaphoreType.DMA((NBUF,)),
            pltpu.SemaphoreType.DMA((NBUF,)),
        ],
    )
    out = run(pe_t, pe_h, pe_w, tf, hf, wf)
    return out.reshape(b, l, D)
